# baseline (speedup 1.0000x reference)
import dataclasses
import functools

import jax
import jax.numpy as jnp
from jax import lax
from jax.experimental import pallas as pl
from jax.experimental.pallas import tpu as pltpu
from jax.experimental.pallas import tpu_sc as plsc

_LAM = 0.2
_NINF = float("-inf")
_L = 16
_NP = 196
_PAD = 208
_CH = _PAD // _L
_K = 98
_OUT = _K + 1
_G1 = 48
_G2 = 64


_BB = 8


def _gram_body(x_ref, s_ref, g_ref):
    xt = jnp.transpose(x_ref[...], (1, 0, 2))
    i = pl.program_id(0)
    srows = s_ref[pl.ds(i * _BB, _BB), :]
    spad = jnp.concatenate(
        [srows, jnp.full((_BB, _PAD - _NP), _NINF, jnp.float32)], axis=1)
    for bb in range(_BB):
        e = xt[bb, 1:, :]
        s = jnp.sum(e * e, axis=1, keepdims=True)
        n = jnp.sqrt(s)
        emb = e / jnp.maximum(n, 1e-12)
        embp = jnp.concatenate(
            [emb, jnp.zeros((_PAD - _NP, emb.shape[1]), jnp.float32)],
            axis=0)
        gram = lax.dot_general(emb, embp, (((1,), (1,)), ((), ())),
                               preferred_element_type=jnp.float32)
        g_ref[bb] = jnp.concatenate(
            [gram, spad[bb:bb + 1, :]], axis=0)


def _gram(xt3, scores):
    b, n = scores.shape[0], _NP + 1
    d = xt3.shape[2]
    return pl.pallas_call(
        _gram_body,
        grid=(b // _BB,),
        in_specs=[pl.BlockSpec((n, _BB, d), lambda i: (0, i, 0)),
                  pl.BlockSpec((b, _NP), lambda i: (0, 0))],
        out_specs=pl.BlockSpec((_BB, n, _PAD), lambda i: (i, 0, 0)),
        out_shape=jax.ShapeDtypeStruct((b, n, _PAD), jnp.float32),
    )(xt3, scores)


def _sel_body(g_hbm, x_hbm, out_hbm, g_v, curr_v, sel_v, idx_v, oidx1_v,
              oidx2_v, rows_v, sem):
    wid = lax.axis_index("s") * 2 + lax.axis_index("c")

    @pl.when(wid < 16)
    def _():
        b = wid
        with jax.named_scope("gdma"):
            pltpu.sync_copy(g_hbm.at[b], g_v)

        lane = lax.iota(jnp.int32, _L)
        poscs = [jnp.int32(c * _L) + lane for c in range(_CH)]
        lane0 = lane == 0
        ninfv = jnp.full((_L,), _NINF, jnp.float32)
        onev = jnp.full((_L,), 1, jnp.int32)

        srow = jnp.full((_L,), _NP, jnp.int32)
        for c in range(_CH):
            curr_v[pl.ds(c * _L, _L)] = plsc.load_gather(
                g_v, [srow, poscs[c]])
            sel_v[pl.ds(c * _L, _L)] = jnp.zeros((_L,), jnp.int32)

        def argmax(obj):
            pairs = [(obj[c], poscs[c]) for c in range(_CH)]
            while len(pairs) > 1:
                nxt = []
                for i in range(0, len(pairs) - 1, 2):
                    (av, ap), (bv2, bp2) = pairs[i], pairs[i + 1]
                    gt = bv2 > av
                    nxt.append((lax.select(gt, bv2, av),
                                lax.select(gt, bp2, ap)))
                if len(pairs) % 2:
                    nxt.append(pairs[-1])
                pairs = nxt
            bv, bp = pairs[0]
            mval = jnp.max(bv)
            cand = bv == jnp.full((_L,), 0.0, jnp.float32) + mval
            posm = lax.select(cand, bp, jnp.full((_L,), 2 * _PAD, jnp.int32))
            return jnp.min(posm)

        def pick(best, msim, first):
            bestv = jnp.full((_L,), 0, jnp.int32) + best
            plsc.store_scatter(curr_v, [bestv], ninfv, mask=lane0)
            plsc.store_scatter(sel_v, [bestv], onev, mask=lane0)
            nmsim = []
            for c in range(_CH):
                row = plsc.load_gather(g_v, [bestv, poscs[c]])
                nmsim.append(row if first else jnp.maximum(msim[c], row))
            return nmsim

        with jax.named_scope("selloop"):
            curr0 = [curr_v[pl.ds(c * _L, _L)] for c in range(_CH)]
            best0 = argmax(curr0)
            msim = pick(best0, None, True)

            def body(_, msim):
                obj = [curr_v[pl.ds(c * _L, _L)] - _LAM * msim[c]
                       for c in range(_CH)]
                best = argmax(obj)
                return pick(best, msim, False)

            msim = lax.fori_loop(1, _K, body, msim)

        bv = jnp.full((_L,), 0, jnp.int32) + b
        for c in range((_G1 + _G2) // _L):
            idx_v[pl.ds(c * _L, _L)] = bv
        off = jnp.int32(1)
        for c in range(_CH):
            selc = sel_v[pl.ds(c * _L, _L)]
            m = selc > 0
            pc = plsc.cumsum(selc)
            posv = off + pc - 1
            val = (poscs[c] + 1) * _L + bv
            plsc.store_scatter(idx_v, [posv], val, mask=m)
            off = off + jnp.sum(selc)

        for c in range(_G1 // _L):
            jv = jnp.int32(c * _L) + lane
            oidx1_v[pl.ds(c * _L, _L)] = jv * _L + bv
        for c in range(_G2 // _L):
            jv = jnp.int32(_G1 + c * _L) + lane
            oidx2_v[pl.ds(c * _L, _L)] = lax.select(
                jv < _OUT, jv * _L + bv, bv)

        with jax.named_scope("rowsdma"):
            pltpu.async_copy(x_hbm.at[idx_v.at[pl.ds(0, _G1)]],
                             rows_v.at[pl.ds(0, _G1)], sem).wait()
            pltpu.async_copy(rows_v.at[pl.ds(0, _G1)],
                             out_hbm.at[oidx1_v], sem).wait()
            pltpu.async_copy(x_hbm.at[idx_v.at[pl.ds(_G1, _G2)]],
                             rows_v, sem).wait()
            pltpu.async_copy(rows_v, out_hbm.at[oidx2_v], sem).wait()


def _select(g, xflat, d):
    b = g.shape[0]
    mesh = plsc.VectorSubcoreMesh(
        core_axis_name="c", subcore_axis_name="s", num_cores=2,
        num_subcores=16)
    cp = pltpu.CompilerParams()
    if "needs_layout_passes" in pltpu.CompilerParams.__dataclass_fields__:
        cp = dataclasses.replace(cp, needs_layout_passes=False)
    run = functools.partial(
        pl.kernel,
        compiler_params=cp,
        out_type=jax.ShapeDtypeStruct((_OUT * b, d), jnp.float32),
        mesh=mesh,
        scratch_types=[
            pltpu.VMEM((_NP + 1, _PAD), jnp.float32),
            pltpu.VMEM((_PAD,), jnp.float32),
            pltpu.VMEM((_PAD,), jnp.int32),
            pltpu.VMEM((_G1 + _G2,), jnp.int32),
            pltpu.VMEM((_G1,), jnp.int32),
            pltpu.VMEM((_G2,), jnp.int32),
            pltpu.VMEM((_G2, d), jnp.float32),
            pltpu.SemaphoreType.DMA,
        ],
    )(_sel_body)
    return run(g, xflat)


def kernel(x, scores):
    b, n, d = x.shape
    xt = x.transpose(1, 0, 2)
    g = _gram(xt, scores)
    out2 = _select(g, xt.reshape(n * b, d), d)
    return out2.reshape(_OUT, b, d).transpose(1, 0, 2)


# speedup vs baseline: 1.0197x; 1.0197x over previous
import dataclasses
import functools

import jax
import jax.numpy as jnp
from jax import lax
from jax.experimental import pallas as pl
from jax.experimental.pallas import tpu as pltpu
from jax.experimental.pallas import tpu_sc as plsc

_LAM = 0.2
_NINF = float("-inf")
_L = 16
_NP = 196
_PAD = 208
_CH = _PAD // _L
_K = 98
_OUT = _K + 1
_G1 = 48
_G2 = 64
_NSLOT = _G1 + _G2


_BB = 8


def _gram_body(x_ref, s_ref, g_ref):
    xt = jnp.transpose(x_ref[...], (1, 0, 2))
    i = pl.program_id(0)
    srows = s_ref[pl.ds(i * _BB, _BB), :]
    spad = jnp.concatenate(
        [srows, jnp.full((_BB, _PAD - _NP), _NINF, jnp.float32)], axis=1)
    for bb in range(_BB):
        e = xt[bb, 1:, :]
        s = jnp.sum(e * e, axis=1, keepdims=True)
        n = jnp.sqrt(s)
        emb = e / jnp.maximum(n, 1e-12)
        embp = jnp.concatenate(
            [emb, jnp.zeros((_PAD - _NP, emb.shape[1]), jnp.float32)],
            axis=0)
        gram = lax.dot_general(emb, embp, (((1,), (1,)), ((), ())),
                               preferred_element_type=jnp.float32)
        g_ref[bb] = jnp.concatenate(
            [gram, spad[bb:bb + 1, :]], axis=0)


def _gram(xt3, scores):
    b, n = scores.shape[0], _NP + 1
    d = xt3.shape[2]
    return pl.pallas_call(
        _gram_body,
        grid=(b // _BB,),
        in_specs=[pl.BlockSpec((n, _BB, d), lambda i: (0, i, 0)),
                  pl.BlockSpec((b, _NP), lambda i: (0, 0))],
        out_specs=pl.BlockSpec((_BB, n, _PAD), lambda i: (i, 0, 0)),
        out_shape=jax.ShapeDtypeStruct((b, n, _PAD), jnp.float32),
    )(xt3, scores)


def _sel_body(g_hbm, x_hbm, out_hbm, curr_v, sel_v, idx_v, oidx1_v,
              oidx2_v, sem, sem2):
    wid = lax.axis_index("s") * 2 + lax.axis_index("c")

    @pl.when(wid < 16)
    def _():
        b = wid
        lane = lax.iota(jnp.int32, _L)
        poscs = [jnp.int32(c * _L) + lane for c in range(_CH)]
        lane0 = lane == 0
        ninfv = jnp.full((_L,), _NINF, jnp.float32)
        onev = jnp.full((_L,), 1, jnp.int32)

        def argmax(obj):
            pairs = [(obj[c], poscs[c]) for c in range(_CH)]
            while len(pairs) > 1:
                nxt = []
                for i in range(0, len(pairs) - 1, 2):
                    (av, ap), (bv2, bp2) = pairs[i], pairs[i + 1]
                    gt = bv2 > av
                    nxt.append((lax.select(gt, bv2, av),
                                lax.select(gt, bp2, ap)))
                if len(pairs) % 2:
                    nxt.append(pairs[-1])
                pairs = nxt
            bv, bp = pairs[0]
            mval = jnp.max(bv)
            cand = bv == jnp.full((_L,), 0.0, jnp.float32) + mval
            posm = lax.select(cand, bp, jnp.full((_L,), 2 * _PAD, jnp.int32))
            return jnp.min(posm)

        def sel_phase(g_v):
            with jax.named_scope("gdma"):
                pltpu.sync_copy(g_hbm.at[b], g_v)

            srow = jnp.full((_L,), _NP, jnp.int32)
            for c in range(_CH):
                curr_v[pl.ds(c * _L, _L)] = plsc.load_gather(
                    g_v, [srow, poscs[c]])
                sel_v[pl.ds(c * _L, _L)] = jnp.zeros((_L,), jnp.int32)

            def pick(best, msim, first):
                bestv = jnp.full((_L,), 0, jnp.int32) + best
                plsc.store_scatter(curr_v, [bestv], ninfv, mask=lane0)
                plsc.store_scatter(sel_v, [bestv], onev, mask=lane0)
                nmsim = []
                for c in range(_CH):
                    row = plsc.load_gather(g_v, [bestv, poscs[c]])
                    nmsim.append(row if first else jnp.maximum(msim[c], row))
                return nmsim

            with jax.named_scope("selloop"):
                curr0 = [curr_v[pl.ds(c * _L, _L)] for c in range(_CH)]
                best0 = argmax(curr0)
                msim = pick(best0, None, True)

                def body(_, msim):
                    obj = [curr_v[pl.ds(c * _L, _L)] - _LAM * msim[c]
                           for c in range(_CH)]
                    best = argmax(obj)
                    return pick(best, msim, False)

                lax.fori_loop(1, _K, body, msim)

        pl.run_scoped(sel_phase, pltpu.VMEM((_NP + 1, _PAD), jnp.float32))

        bv = jnp.full((_L,), 0, jnp.int32) + b
        for c in range(_NSLOT // _L):
            idx_v[pl.ds(c * _L, _L)] = bv
        off = jnp.int32(1)
        for c in range(_CH):
            selc = sel_v[pl.ds(c * _L, _L)]
            m = selc > 0
            pc = plsc.cumsum(selc)
            posv = off + pc - 1
            val = (poscs[c] + 1) * _L + bv
            plsc.store_scatter(idx_v, [posv], val, mask=m)
            off = off + jnp.sum(selc)

        for c in range(_G1 // _L):
            jv = jnp.int32(c * _L) + lane
            oidx1_v[pl.ds(c * _L, _L)] = jv * _L + bv
        for c in range(_G2 // _L):
            jv = jnp.int32(_G1 + c * _L) + lane
            oidx2_v[pl.ds(c * _L, _L)] = lax.select(
                jv < _OUT, jv * _L + bv, bv)

        def move_phase(r1_v, r2_v):
            with jax.named_scope("rowsdma"):
                ga = pltpu.async_copy(x_hbm.at[idx_v.at[pl.ds(0, _G1)]],
                                      r1_v, sem)
                gb = pltpu.async_copy(x_hbm.at[idx_v.at[pl.ds(_G1, _G2)]],
                                      r2_v, sem2)
                ga.wait()
                gb.wait()
                sa = pltpu.async_copy(r1_v, out_hbm.at[oidx1_v], sem)
                sb = pltpu.async_copy(r2_v, out_hbm.at[oidx2_v], sem2)
                sa.wait()
                sb.wait()

        pl.run_scoped(move_phase,
                      pltpu.VMEM((_G1, 768), jnp.float32),
                      pltpu.VMEM((_G2, 768), jnp.float32))


def _select(g, xflat, d):
    b = g.shape[0]
    mesh = plsc.VectorSubcoreMesh(
        core_axis_name="c", subcore_axis_name="s", num_cores=2,
        num_subcores=16)
    cp = pltpu.CompilerParams()
    if "needs_layout_passes" in pltpu.CompilerParams.__dataclass_fields__:
        cp = dataclasses.replace(cp, needs_layout_passes=False)
    run = functools.partial(
        pl.kernel,
        compiler_params=cp,
        out_type=jax.ShapeDtypeStruct((_OUT * b, d), jnp.float32),
        mesh=mesh,
        scratch_types=[
            pltpu.VMEM((_PAD,), jnp.float32),
            pltpu.VMEM((_PAD,), jnp.int32),
            pltpu.VMEM((_NSLOT,), jnp.int32),
            pltpu.VMEM((_G1,), jnp.int32),
            pltpu.VMEM((_G2,), jnp.int32),
            pltpu.SemaphoreType.DMA,
            pltpu.SemaphoreType.DMA,
        ],
    )(_sel_body)
    return run(g, xflat)


def kernel(x, scores):
    b, n, d = x.shape
    xt = x.transpose(1, 0, 2)
    g = _gram(xt, scores)
    out2 = _select(g, xt.reshape(n * b, d), d)
    return out2.reshape(_OUT, b, d).transpose(1, 0, 2)
